# baseline (speedup 1.0000x reference)
import jax
import jax.numpy as jnp
from jax import lax
from jax.experimental import pallas as pl
from jax.experimental.pallas import tpu as pltpu
from jax.experimental.pallas import tpu_sc as plsc

N = 10000
E = 320000
NC = 2
NS = 16
CH = 64
NG = 8
NBUF = 4
LA = 2
NSLOT = 3
NT = N + 16
RB = 624
RB_EX0 = NS * RB
HR = 80
OUT_CH = 2

_MESH = plsc.VectorSubcoreMesh(
    core_axis_name="c", subcore_axis_name="s", num_cores=NC, num_subcores=NS)


def _leaky(v):
  return jnp.where(v >= 0, v, 0.01 * v)


def _sc_agg(table, srcA, srcB, dst_g, z128, with_deg):
  edge_split = srcA.shape[0] == NC * NS
  n_g = srcA.shape[1] - 1

  out_type = [jax.ShapeDtypeStruct((NC, N, 128), jnp.float32)]
  if with_deg:
    out_type.append(jax.ShapeDtypeStruct((NC * NS, HR, 128), jnp.float32))

  scratch = [
      pltpu.VMEM((NSLOT, NG, CH), jnp.int32),
      pltpu.VMEM((NSLOT, NG, CH), jnp.int32),
  ]
  scratch += [pltpu.VMEM((CH, 128), jnp.float32) for _ in range(NBUF)]
  if with_deg:
    scratch.append(pltpu.VMEM((HR, 128), jnp.float32))
  scratch.append(pltpu.VMEM_SHARED((NT, 128), jnp.float32))
  scratch += [pltpu.SemaphoreType.DMA] * (2 * NBUF + 1)

  def body(table_h, srcA_h, srcB_h, dst_h, z128_h, *rest):
    if with_deg:
      out_h, deg_h = rest[0], rest[1]
      rest = rest[2:]
    else:
      out_h = rest[0]
      deg_h = None
      rest = rest[1:]
    src_v, dst_v = rest[0], rest[1]
    bufs = rest[2:2 + NBUF]
    rest = rest[2 + NBUF:]
    if with_deg:
      hist_v = rest[0]
      rest = rest[1:]
    else:
      hist_v = None
    agg_s = rest[0]
    gsems = rest[1:1 + NBUF]
    ssems = rest[1 + NBUF:1 + 2 * NBUF]
    semi = rest[1 + 2 * NBUF]

    c = lax.axis_index("c")
    s = lax.axis_index("s")
    w = c * NS + s if edge_split else s
    r0 = s * RB

    pltpu.sync_copy(z128_h, agg_s.at[pl.ds(r0, RB)])
    if with_deg:
      pltpu.sync_copy(z128_h.at[pl.ds(0, HR)], hist_v)

    @pl.when(s == NS - 1)
    def _():
      pltpu.sync_copy(z128_h.at[pl.ds(0, NT - RB_EX0)],
                      agg_s.at[pl.ds(RB_EX0, NT - RB_EX0)])

    def fetch_idx(g, slot):
      @pl.when(c == 0)
      def _():
        pltpu.async_copy(srcA_h.at[w, g], src_v.at[slot], semi)

      @pl.when(c == 1)
      def _():
        pltpu.async_copy(srcB_h.at[w, g], src_v.at[slot], semi)

      pltpu.async_copy(dst_h.at[w, g], dst_v.at[slot], semi)

    def wait_idx(g, slot):
      pltpu.make_async_copy(srcA_h.at[w, g], src_v.at[slot], semi).wait()
      pltpu.make_async_copy(dst_h.at[w, g], dst_v.at[slot], semi).wait()

    def start_gather(slot, k, b):
      pltpu.async_copy(table_h.at[src_v.at[slot, k]], bufs[b], gsems[b])

    def wait_gather(slot, k, b):
      pltpu.make_async_copy(table_h.at[src_v.at[slot, k]], bufs[b],
                            gsems[b]).wait()

    def start_scatter(slot, k, b):
      pltpu.async_copy(bufs[b], agg_s.at[dst_v.at[slot, k]], ssems[b],
                       add=True)

    def wait_scatter(slot, k, b):
      pltpu.make_async_copy(bufs[b], agg_s.at[dst_v.at[slot, k]],
                            ssems[b]).wait()

    fetch_idx(0, 0)
    plsc.subcore_barrier()
    wait_idx(0, 0)
    for j in range(LA):
      start_gather(0, j, j)

    ones16 = jnp.ones((16,), jnp.float32)

    @pl.loop(0, n_g)
    def _(g):
      sl = lax.rem(g, NSLOT)
      sl_n = lax.rem(g + 1, NSLOT)
      fetch_idx(g + 1, sl_n)
      for k in range(NG):
        b = k % NBUF
        b_i = (k + LA) % NBUF
        if k + LA >= NBUF:
          wait_scatter(sl, k + LA - NBUF, b_i)
        else:
          @pl.when(g > 0)
          def _():
            wait_scatter(lax.rem(g + NSLOT - 1, NSLOT),
                         k + LA - NBUF + NG, b_i)
        if k + LA < NG:
          start_gather(sl, k + LA, b_i)
        else:
          if k == NG - LA:
            wait_idx(g + 1, sl_n)
          start_gather(sl_n, k + LA - NG, b_i)
        wait_gather(sl, k, b)
        start_scatter(sl, k, b)
        if with_deg:
          for q in range(CH // 16):
            v = dst_v[sl, k, pl.ds(q * 16, 16)]
            plsc.addupdate_scatter(
                hist_v, [lax.shift_right_logical(v, 7),
                         lax.bitwise_and(v, 127)], ones16)

    sl_last = lax.rem(n_g - 1, NSLOT)
    sl_dummy = lax.rem(n_g, NSLOT)
    for j in range(LA):
      wait_gather(sl_dummy, j, j)
    for j in range(NBUF - LA):
      k = NG - (NBUF - LA) + j
      wait_scatter(sl_last, k, k % NBUF)

    plsc.subcore_barrier()
    pltpu.sync_copy(agg_s.at[pl.ds(r0, RB)], out_h.at[c, pl.ds(r0, RB)])
    if with_deg:
      pltpu.sync_copy(hist_v, deg_h.at[c * NS + s])

    @pl.when(s == NS - 1)
    def _():
      pltpu.sync_copy(agg_s.at[pl.ds(RB_EX0, N - RB_EX0)],
                      out_h.at[c, pl.ds(RB_EX0, N - RB_EX0)])

  fn = pl.kernel(
      body, out_type=tuple(out_type), mesh=_MESH, scratch_types=scratch,
      compiler_params=pltpu.CompilerParams(needs_layout_passes=False))
  return fn(table, srcA, srcB, dst_g, z128)


_R = 1000
_GRID = N // _R


def _row_spec(d):
  return pl.BlockSpec((_R, d), lambda i: (i, 0))


def _pair_spec(d):
  return pl.BlockSpec((NC, _R, d), lambda i: (0, i, 0))


def _full_spec(r, c):
  return pl.BlockSpec((r, c), lambda i: (0, 0))


def _tc_deg_prep(degs):
  def body(d_ref, o_ref):
    hs = jnp.sum(d_ref[...], axis=0)
    deg = hs.reshape(HR * 128)[:N]
    di = 1.0 / jnp.maximum(deg, 1.0)
    o_ref[...] = jnp.broadcast_to(di[:, None], (N, 128))

  return pl.pallas_call(
      body,
      grid=(1,),
      in_specs=[pl.BlockSpec((NC * NS, HR, 128), lambda i: (0, 0, 0))],
      out_specs=pl.BlockSpec((N, 128), lambda i: (0, 0)),
      out_shape=jax.ShapeDtypeStruct((N, 128), jnp.float32),
  )(degs)


def _tc_layer0(x, s0, di, W1, b1, W2, b2, eps):
  def body(x_ref, s0_ref, di_ref, W1_ref, b1_ref, W2_ref, b2_ref, eps_ref,
           h1_ref):
    agg = (s0_ref[0] + s0_ref[1]) * di_ref[...]
    z = (1.0 + eps_ref[0, 0]) * x_ref[...] + agg
    a = _leaky(jnp.dot(z, W1_ref[...], preferred_element_type=jnp.float32)
               + b1_ref[...])
    h1_ref[...] = _leaky(
        jnp.dot(a, W2_ref[...], preferred_element_type=jnp.float32)
        + b2_ref[...])

  return pl.pallas_call(
      body,
      grid=(_GRID,),
      in_specs=[
          _row_spec(128), _pair_spec(128), _row_spec(128),
          _full_spec(128, 256), _full_spec(1, 256),
          _full_spec(256, 256), _full_spec(1, 256),
          _full_spec(1, 1),
      ],
      out_specs=_row_spec(256),
      out_shape=jax.ShapeDtypeStruct((N, 256), jnp.float32),
  )(x, s0, di, W1, b1, W2, b2, eps)


def _tc_layer1(h1, s1, di, W1, b1, W2, b2, Wp, eps):
  def body(h1_ref, s1_ref, di_ref, W1_ref, b1_ref, W2_ref, b2_ref,
           Wp_ref, eps_ref, p_ref):
    agg = jnp.concatenate([s1_ref[0], s1_ref[1]], axis=1) * di_ref[...][:, :1]
    z = (1.0 + eps_ref[0, 0]) * h1_ref[...] + agg
    a = _leaky(jnp.dot(z, W1_ref[...], preferred_element_type=jnp.float32)
               + b1_ref[...])
    h2 = _leaky(jnp.dot(a, W2_ref[...], preferred_element_type=jnp.float32)
                + b2_ref[...])
    p_ref[...] = jnp.dot(h2, Wp_ref[...], preferred_element_type=jnp.float32)

  return pl.pallas_call(
      body,
      grid=(_GRID,),
      in_specs=[
          _row_spec(256), _pair_spec(128), _row_spec(128),
          _full_spec(256, 256), _full_spec(1, 256),
          _full_spec(256, 256), _full_spec(1, 256),
          _full_spec(256, 128),
          _full_spec(1, 1),
      ],
      out_specs=_row_spec(128),
      out_shape=jax.ShapeDtypeStruct((N, 128), jnp.float32),
  )(h1, s1, di, W1, b1, W2, b2, Wp, eps)


def _tc_layer2(p, s2, di, b1, W2, b2, Wo, bo, eps):
  def body(p_ref, s2_ref, di_ref, b1_ref, W2_ref, b2_ref, Wo_ref, bo_ref,
           eps_ref, no_ref, ne_ref):
    agg = (s2_ref[0] + s2_ref[1]) * di_ref[...]
    z = (1.0 + eps_ref[0, 0]) * p_ref[...] + agg + b1_ref[...]
    a = _leaky(z)
    ne = _leaky(jnp.dot(a, W2_ref[...], preferred_element_type=jnp.float32)
                + b2_ref[...])
    ne_ref[...] = ne
    no_ref[...] = (jnp.dot(ne, Wo_ref[...], preferred_element_type=jnp.float32)
                   + bo_ref[...])

  return pl.pallas_call(
      body,
      grid=(_GRID,),
      in_specs=[
          _row_spec(128), _pair_spec(128), _row_spec(128),
          _full_spec(1, 128),
          _full_spec(128, 128), _full_spec(1, 128),
          _full_spec(128, 128), _full_spec(1, 128),
          _full_spec(1, 1),
      ],
      out_specs=(_row_spec(128), _row_spec(128)),
      out_shape=(jax.ShapeDtypeStruct((N, 128), jnp.float32),
                 jax.ShapeDtypeStruct((N, 128), jnp.float32)),
  )(p, s2, di, b1, W2, b2, Wo, bo, eps)


def _chunk(a, workers, fill):
  per = E // workers
  a = a.reshape(workers, per)
  pad = (-per) % (NG * CH) + NG * CH
  a = jnp.concatenate(
      [a, jnp.full((workers, pad), fill, dtype=a.dtype)], axis=1)
  return a.reshape(workers, -1, NG, CH)


def kernel(x, edge_index,
           W1_0, b1_0, W2_0, b2_0,
           W1_1, b1_1, W2_1, b2_1,
           W1_2, b1_2, W2_2, b2_2,
           Wout, bout, eps0, eps1, eps2):
  src = edge_index[0].astype(jnp.int32)
  dst = edge_index[1].astype(jnp.int32)

  src_e = _chunk(src, NC * NS, 0)
  dst_e = _chunk(dst, NC * NS, N)
  srcA = _chunk(src * 2, NS, 0)
  srcB = _chunk(src * 2 + 1, NS, 1)
  dst_c = _chunk(dst, NS, N)

  z128 = jnp.zeros((RB, 128), jnp.float32)

  def r2(b):
    return b.reshape(1, -1)

  def e2(e):
    return e.astype(jnp.float32).reshape(1, 1)

  s0, degs = _sc_agg(x, src_e, src_e, dst_e, z128, with_deg=True)
  di = _tc_deg_prep(degs)
  h1 = _tc_layer0(x, s0, di, W1_0, r2(b1_0), W2_0, r2(b2_0), e2(eps0))

  s1 = _sc_agg(h1.reshape(2 * N, 128), srcA, srcB, dst_c, z128,
               with_deg=False)[0]
  p = _tc_layer1(h1, s1, di, W1_1, r2(b1_1), W2_1, r2(b2_1), W1_2, e2(eps1))

  s2 = _sc_agg(p, src_e, src_e, dst_e, z128, with_deg=False)[0]
  Wo = jnp.pad(Wout, ((0, 0), (0, 128 - OUT_CH)))
  bo = jnp.pad(bout, (0, 128 - OUT_CH)).reshape(1, -1)
  n_out_pad, n_embed = _tc_layer2(p, s2, di, r2(b1_2), W2_2, r2(b2_2),
                                  Wo, bo, e2(eps2))
  return (n_out_pad[:, :OUT_CH], n_embed)


# speedup vs baseline: 1.5081x; 1.5081x over previous
import jax
import jax.numpy as jnp
from jax import lax
from jax.experimental import pallas as pl
from jax.experimental.pallas import tpu as pltpu
from jax.experimental.pallas import tpu_sc as plsc

N = 10000
E = 320000
NC = 2
NS = 16
CH = 128
NG = 8
NT = N + 16
RB = 624
RB_EX0 = NS * RB
HR = 80
OUT_CH = 2

_MESH = plsc.VectorSubcoreMesh(
    core_axis_name="c", subcore_axis_name="s", num_cores=NC, num_subcores=NS)


def _leaky(v):
  return jnp.where(v >= 0, v, 0.01 * v)


def _sc_agg(table, srcA, srcB, dst_g, z128, with_deg):
  edge_split = srcA.shape[0] == NC * NS
  n_g = srcA.shape[1] - 1

  out_type = [jax.ShapeDtypeStruct((NC, N, 128), jnp.float32)]
  if with_deg:
    out_type.append(jax.ShapeDtypeStruct((NC * NS, HR, 128), jnp.float32))

  scratch = [
      pltpu.VMEM((2, NG, CH), jnp.int32),
      pltpu.VMEM((2, NG, CH), jnp.int32),
      pltpu.VMEM((CH, 128), jnp.float32),
      pltpu.VMEM((CH, 128), jnp.float32),
      pltpu.VMEM((HR, 128), jnp.float32) if with_deg else None,
      pltpu.VMEM_SHARED((NT, 128), jnp.float32),
      pltpu.SemaphoreType.DMA,
      pltpu.SemaphoreType.DMA,
      pltpu.SemaphoreType.DMA,
  ]
  scratch = [sc for sc in scratch if sc is not None]

  def body(table_h, srcA_h, srcB_h, dst_h, z128_h, *rest):
    if with_deg:
      out_h, deg_h = rest[0], rest[1]
      (src_v, dst_v, buf0, buf1, hist_v, agg_s, sem0, sem1, semi) = rest[2:]
    else:
      out_h = rest[0]
      deg_h = hist_v = None
      (src_v, dst_v, buf0, buf1, agg_s, sem0, sem1, semi) = rest[1:]

    c = lax.axis_index("c")
    s = lax.axis_index("s")
    w = c * NS + s if edge_split else s
    r0 = s * RB

    pltpu.sync_copy(z128_h, agg_s.at[pl.ds(r0, RB)])
    if with_deg:
      pltpu.sync_copy(z128_h.at[pl.ds(0, HR)], hist_v)

    @pl.when(s == NS - 1)
    def _():
      pltpu.sync_copy(z128_h.at[pl.ds(0, NT - RB_EX0)],
                      agg_s.at[pl.ds(RB_EX0, NT - RB_EX0)])

    def fetch_idx(g, slot):
      @pl.when(c == 0)
      def _():
        pltpu.async_copy(srcA_h.at[w, g], src_v.at[slot], semi)

      @pl.when(c == 1)
      def _():
        pltpu.async_copy(srcB_h.at[w, g], src_v.at[slot], semi)

      pltpu.async_copy(dst_h.at[w, g], dst_v.at[slot], semi)

    def wait_idx(g, slot):
      pltpu.make_async_copy(srcA_h.at[w, g], src_v.at[slot], semi).wait()
      pltpu.make_async_copy(dst_h.at[w, g], dst_v.at[slot], semi).wait()

    fetch_idx(0, 0)
    plsc.subcore_barrier()

    ones16 = jnp.ones((16,), jnp.float32)
    bufs = (buf0, buf1)
    sems = (sem0, sem1)

    @pl.loop(0, n_g)
    def _(g):
      p = lax.rem(g, 2)
      wait_idx(g, p)
      fetch_idx(g + 1, 1 - p)
      cps = [pltpu.async_copy(table_h.at[src_v.at[p, 0]], buf0, sem0)]
      for k in range(NG):
        if k + 1 < NG:
          cps.append(pltpu.async_copy(
              table_h.at[src_v.at[p, k + 1]], bufs[(k + 1) % 2],
              sems[(k + 1) % 2]))
        cps[k].wait()
        pltpu.sync_copy(bufs[k % 2], agg_s.at[dst_v.at[p, k]], add=True)
        if with_deg:
          for q in range(CH // 16):
            v = dst_v[p, k, pl.ds(q * 16, 16)]
            plsc.addupdate_scatter(
                hist_v, [lax.shift_right_logical(v, 7),
                         lax.bitwise_and(v, 127)], ones16)

    wait_idx(n_g, n_g % 2)

    plsc.subcore_barrier()
    pltpu.sync_copy(agg_s.at[pl.ds(r0, RB)], out_h.at[c, pl.ds(r0, RB)])
    if with_deg:
      pltpu.sync_copy(hist_v, deg_h.at[c * NS + s])

    @pl.when(s == NS - 1)
    def _():
      pltpu.sync_copy(agg_s.at[pl.ds(RB_EX0, N - RB_EX0)],
                      out_h.at[c, pl.ds(RB_EX0, N - RB_EX0)])

  fn = pl.kernel(
      body, out_type=tuple(out_type), mesh=_MESH, scratch_types=scratch,
      compiler_params=pltpu.CompilerParams(needs_layout_passes=False))
  return fn(table, srcA, srcB, dst_g, z128)


_R = 1000
_GRID = N // _R


def _row_spec(d):
  return pl.BlockSpec((_R, d), lambda i: (i, 0))


def _pair_spec(d):
  return pl.BlockSpec((NC, _R, d), lambda i: (0, i, 0))


def _full_spec(r, c):
  return pl.BlockSpec((r, c), lambda i: (0, 0))


def _tc_deg_prep(degs):
  def body(d_ref, o_ref):
    hs = jnp.sum(d_ref[...], axis=0)
    deg = hs.reshape(HR * 128)[:N]
    di = 1.0 / jnp.maximum(deg, 1.0)
    o_ref[...] = jnp.broadcast_to(di[:, None], (N, 128))

  return pl.pallas_call(
      body,
      grid=(1,),
      in_specs=[pl.BlockSpec((NC * NS, HR, 128), lambda i: (0, 0, 0))],
      out_specs=pl.BlockSpec((N, 128), lambda i: (0, 0)),
      out_shape=jax.ShapeDtypeStruct((N, 128), jnp.float32),
  )(degs)


def _tc_layer0(x, s0, di, W1, b1, W2, b2, eps):
  def body(x_ref, s0_ref, di_ref, W1_ref, b1_ref, W2_ref, b2_ref, eps_ref,
           h1_ref):
    agg = (s0_ref[0] + s0_ref[1]) * di_ref[...]
    z = (1.0 + eps_ref[0, 0]) * x_ref[...] + agg
    a = _leaky(jnp.dot(z, W1_ref[...], preferred_element_type=jnp.float32)
               + b1_ref[...])
    h1_ref[...] = _leaky(
        jnp.dot(a, W2_ref[...], preferred_element_type=jnp.float32)
        + b2_ref[...])

  return pl.pallas_call(
      body,
      grid=(_GRID,),
      in_specs=[
          _row_spec(128), _pair_spec(128), _row_spec(128),
          _full_spec(128, 256), _full_spec(1, 256),
          _full_spec(256, 256), _full_spec(1, 256),
          _full_spec(1, 1),
      ],
      out_specs=_row_spec(256),
      out_shape=jax.ShapeDtypeStruct((N, 256), jnp.float32),
  )(x, s0, di, W1, b1, W2, b2, eps)


def _tc_layer1(h1, s1, di, W1, b1, W2, b2, Wp, eps):
  def body(h1_ref, s1_ref, di_ref, W1_ref, b1_ref, W2_ref, b2_ref,
           Wp_ref, eps_ref, p_ref):
    agg = jnp.concatenate([s1_ref[0], s1_ref[1]], axis=1) * di_ref[...][:, :1]
    z = (1.0 + eps_ref[0, 0]) * h1_ref[...] + agg
    a = _leaky(jnp.dot(z, W1_ref[...], preferred_element_type=jnp.float32)
               + b1_ref[...])
    h2 = _leaky(jnp.dot(a, W2_ref[...], preferred_element_type=jnp.float32)
                + b2_ref[...])
    p_ref[...] = jnp.dot(h2, Wp_ref[...], preferred_element_type=jnp.float32)

  return pl.pallas_call(
      body,
      grid=(_GRID,),
      in_specs=[
          _row_spec(256), _pair_spec(128), _row_spec(128),
          _full_spec(256, 256), _full_spec(1, 256),
          _full_spec(256, 256), _full_spec(1, 256),
          _full_spec(256, 128),
          _full_spec(1, 1),
      ],
      out_specs=_row_spec(128),
      out_shape=jax.ShapeDtypeStruct((N, 128), jnp.float32),
  )(h1, s1, di, W1, b1, W2, b2, Wp, eps)


def _tc_layer2(p, s2, di, b1, W2, b2, Wo, bo, eps):
  def body(p_ref, s2_ref, di_ref, b1_ref, W2_ref, b2_ref, Wo_ref, bo_ref,
           eps_ref, no_ref, ne_ref):
    agg = (s2_ref[0] + s2_ref[1]) * di_ref[...]
    z = (1.0 + eps_ref[0, 0]) * p_ref[...] + agg + b1_ref[...]
    a = _leaky(z)
    ne = _leaky(jnp.dot(a, W2_ref[...], preferred_element_type=jnp.float32)
                + b2_ref[...])
    ne_ref[...] = ne
    no_ref[...] = (jnp.dot(ne, Wo_ref[...], preferred_element_type=jnp.float32)
                   + bo_ref[...])

  return pl.pallas_call(
      body,
      grid=(_GRID,),
      in_specs=[
          _row_spec(128), _pair_spec(128), _row_spec(128),
          _full_spec(1, 128),
          _full_spec(128, 128), _full_spec(1, 128),
          _full_spec(128, 128), _full_spec(1, 128),
          _full_spec(1, 1),
      ],
      out_specs=(_row_spec(128), _row_spec(128)),
      out_shape=(jax.ShapeDtypeStruct((N, 128), jnp.float32),
                 jax.ShapeDtypeStruct((N, 128), jnp.float32)),
  )(p, s2, di, b1, W2, b2, Wo, bo, eps)


def _chunk(a, workers, fill):
  per = E // workers
  a = a.reshape(workers, per)
  pad = (-per) % (NG * CH) + NG * CH
  a = jnp.concatenate(
      [a, jnp.full((workers, pad), fill, dtype=a.dtype)], axis=1)
  return a.reshape(workers, -1, NG, CH)


def kernel(x, edge_index,
           W1_0, b1_0, W2_0, b2_0,
           W1_1, b1_1, W2_1, b2_1,
           W1_2, b1_2, W2_2, b2_2,
           Wout, bout, eps0, eps1, eps2):
  src = edge_index[0].astype(jnp.int32)
  dst = edge_index[1].astype(jnp.int32)

  srcA_e = _chunk(src, NC * NS, 0)
  srcB_e = _chunk(src + N, NC * NS, N)
  dst_e = _chunk(dst, NC * NS, N)
  srcA = _chunk(src * 2, NS, 0)
  srcB = _chunk(src * 2 + 1, NS, 1)
  dst_c = _chunk(dst, NS, N)

  z128 = jnp.zeros((RB, 128), jnp.float32)

  def r2(b):
    return b.reshape(1, -1)

  def e2(e):
    return e.astype(jnp.float32).reshape(1, 1)

  s0, degs = _sc_agg(jnp.concatenate([x, x]), srcA_e, srcB_e, dst_e, z128,
                     with_deg=True)
  di = _tc_deg_prep(degs)
  h1 = _tc_layer0(x, s0, di, W1_0, r2(b1_0), W2_0, r2(b2_0), e2(eps0))

  s1 = _sc_agg(h1.reshape(2 * N, 128), srcA, srcB, dst_c, z128,
               with_deg=False)[0]
  p = _tc_layer1(h1, s1, di, W1_1, r2(b1_1), W2_1, r2(b2_1), W1_2, e2(eps1))

  s2 = _sc_agg(jnp.concatenate([p, p]), srcA_e, srcB_e, dst_e, z128,
               with_deg=False)[0]
  Wo = jnp.pad(Wout, ((0, 0), (0, 128 - OUT_CH)))
  bo = jnp.pad(bout, (0, 128 - OUT_CH)).reshape(1, -1)
  n_out_pad, n_embed = _tc_layer2(p, s2, di, r2(b1_2), W2_2, r2(b2_2),
                                  Wo, bo, e2(eps2))
  return (n_out_pad[:, :OUT_CH], n_embed)


# speedup vs baseline: 1.5304x; 1.0148x over previous
import jax
import jax.numpy as jnp
from jax import lax
from jax.experimental import pallas as pl
from jax.experimental.pallas import tpu as pltpu
from jax.experimental.pallas import tpu_sc as plsc

N = 10000
E = 320000
NC = 2
NS = 16
CH = 128
NG = 8
NT = N + 8
RB = 624
RB_EX0 = NS * RB
HR = 80
OUT_CH = 2

_MESH = plsc.VectorSubcoreMesh(
    core_axis_name="c", subcore_axis_name="s", num_cores=NC, num_subcores=NS)


def _leaky(v):
  return jnp.where(v >= 0, v, 0.01 * v)


def _sc_agg(table, srcA, srcB, dst_g, z128, with_deg, nbuf=2, la=1, ng=NG,
            ch=CH):
  assert ng % nbuf == 0 and 0 < la < nbuf and la <= ng
  edge_split = srcA.shape[0] == NC * NS
  n_g = srcA.shape[1] - 1

  out_type = [jax.ShapeDtypeStruct((NC, N, 128), jnp.float32)]
  if with_deg:
    out_type.append(jax.ShapeDtypeStruct((NC * NS, HR, 128), jnp.float32))

  scratch = [
      pltpu.VMEM((2, ng, ch), jnp.int32),
      pltpu.VMEM((2, ng, ch), jnp.int32),
  ]
  scratch += [pltpu.VMEM((ch, 128), jnp.float32) for _ in range(nbuf)]
  if with_deg:
    scratch.append(pltpu.VMEM((HR, 128), jnp.float32))
  scratch.append(pltpu.VMEM_SHARED((NT, 128), jnp.float32))
  scratch += [pltpu.SemaphoreType.DMA] * (nbuf + 1)

  def body(table_h, srcA_h, srcB_h, dst_h, z128_h, *rest):
    if with_deg:
      out_h, deg_h = rest[0], rest[1]
      rest = rest[2:]
    else:
      out_h = rest[0]
      deg_h = None
      rest = rest[1:]
    src_v, dst_v = rest[0], rest[1]
    bufs = rest[2:2 + nbuf]
    rest = rest[2 + nbuf:]
    if with_deg:
      hist_v = rest[0]
      rest = rest[1:]
    else:
      hist_v = None
    agg_s = rest[0]
    gsems = rest[1:1 + nbuf]
    semi = rest[1 + nbuf]

    c = lax.axis_index("c")
    s = lax.axis_index("s")
    w = c * NS + s if edge_split else s
    r0 = s * RB

    pltpu.sync_copy(z128_h, agg_s.at[pl.ds(r0, RB)])
    if with_deg:
      pltpu.sync_copy(z128_h.at[pl.ds(0, HR)], hist_v)

    @pl.when(s == NS - 1)
    def _():
      pltpu.sync_copy(z128_h.at[pl.ds(0, NT - RB_EX0)],
                      agg_s.at[pl.ds(RB_EX0, NT - RB_EX0)])

    def fetch_idx(g, slot):
      @pl.when(c == 0)
      def _():
        pltpu.async_copy(srcA_h.at[w, g], src_v.at[slot], semi)

      @pl.when(c == 1)
      def _():
        pltpu.async_copy(srcB_h.at[w, g], src_v.at[slot], semi)

      pltpu.async_copy(dst_h.at[w, g], dst_v.at[slot], semi)

    def wait_idx(g, slot):
      pltpu.make_async_copy(srcA_h.at[w, g], src_v.at[slot], semi).wait()
      pltpu.make_async_copy(dst_h.at[w, g], dst_v.at[slot], semi).wait()

    def start_gather(slot, k, b):
      pltpu.async_copy(table_h.at[src_v.at[slot, k]], bufs[b], gsems[b])

    def wait_gather(slot, k, b):
      pltpu.make_async_copy(table_h.at[src_v.at[slot, k]], bufs[b],
                            gsems[b]).wait()

    fetch_idx(0, 0)
    plsc.subcore_barrier()
    wait_idx(0, 0)
    for j in range(la):
      start_gather(0, j, j)

    ones16 = jnp.ones((16,), jnp.float32)

    @pl.loop(0, n_g)
    def _(g):
      p = lax.rem(g, 2)
      fetch_idx(g + 1, 1 - p)
      for k in range(ng):
        b = k % nbuf
        if k + la < ng:
          start_gather(p, k + la, (k + la) % nbuf)
        else:
          if k == ng - la:
            wait_idx(g + 1, 1 - p)
          start_gather(1 - p, k + la - ng, (k + la) % nbuf)
        wait_gather(p, k, b)
        pltpu.sync_copy(bufs[b], agg_s.at[dst_v.at[p, k]], add=True)
        if with_deg:
          for q in range(ch // 16):
            v = dst_v[p, k, pl.ds(q * 16, 16)]
            plsc.addupdate_scatter(
                hist_v, [lax.shift_right_logical(v, 7),
                         lax.bitwise_and(v, 127)], ones16)

    p_dummy = lax.rem(n_g, 2)
    for j in range(la):
      wait_gather(p_dummy, j, j)

    plsc.subcore_barrier()
    pltpu.sync_copy(agg_s.at[pl.ds(r0, RB)], out_h.at[c, pl.ds(r0, RB)])
    if with_deg:
      pltpu.sync_copy(hist_v, deg_h.at[c * NS + s])

    @pl.when(s == NS - 1)
    def _():
      pltpu.sync_copy(agg_s.at[pl.ds(RB_EX0, N - RB_EX0)],
                      out_h.at[c, pl.ds(RB_EX0, N - RB_EX0)])

  fn = pl.kernel(
      body, out_type=tuple(out_type), mesh=_MESH, scratch_types=scratch,
      compiler_params=pltpu.CompilerParams(needs_layout_passes=False))
  return fn(table, srcA, srcB, dst_g, z128)


_R = 1000
_GRID = N // _R


def _row_spec(d):
  return pl.BlockSpec((_R, d), lambda i: (i, 0))


def _pair_spec(d):
  return pl.BlockSpec((NC, _R, d), lambda i: (0, i, 0))


def _full_spec(r, c):
  return pl.BlockSpec((r, c), lambda i: (0, 0))


def _tc_deg_prep(degs):
  def body(d_ref, o_ref):
    hs = jnp.sum(d_ref[...], axis=0)
    deg = hs.reshape(HR * 128)[:N]
    di = 1.0 / jnp.maximum(deg, 1.0)
    o_ref[...] = jnp.broadcast_to(di[:, None], (N, 128))

  return pl.pallas_call(
      body,
      grid=(1,),
      in_specs=[pl.BlockSpec((NC * NS, HR, 128), lambda i: (0, 0, 0))],
      out_specs=pl.BlockSpec((N, 128), lambda i: (0, 0)),
      out_shape=jax.ShapeDtypeStruct((N, 128), jnp.float32),
  )(degs)


def _tc_layer0(x, s0, di, W1, b1, W2, b2, eps):
  def body(x_ref, s0_ref, di_ref, W1_ref, b1_ref, W2_ref, b2_ref, eps_ref,
           h1_ref):
    agg = (s0_ref[0] + s0_ref[1]) * di_ref[...]
    z = (1.0 + eps_ref[0, 0]) * x_ref[...] + agg
    a = _leaky(jnp.dot(z, W1_ref[...], preferred_element_type=jnp.float32)
               + b1_ref[...])
    h1_ref[...] = _leaky(
        jnp.dot(a, W2_ref[...], preferred_element_type=jnp.float32)
        + b2_ref[...])

  return pl.pallas_call(
      body,
      grid=(_GRID,),
      in_specs=[
          _row_spec(128), _pair_spec(128), _row_spec(128),
          _full_spec(128, 256), _full_spec(1, 256),
          _full_spec(256, 256), _full_spec(1, 256),
          _full_spec(1, 1),
      ],
      out_specs=_row_spec(256),
      out_shape=jax.ShapeDtypeStruct((N, 256), jnp.float32),
  )(x, s0, di, W1, b1, W2, b2, eps)


def _tc_layer1(h1, s1, di, W1, b1, W2, b2, Wp, eps):
  def body(h1_ref, s1_ref, di_ref, W1_ref, b1_ref, W2_ref, b2_ref,
           Wp_ref, eps_ref, p_ref):
    agg = jnp.concatenate([s1_ref[0], s1_ref[1]], axis=1) * di_ref[...][:, :1]
    z = (1.0 + eps_ref[0, 0]) * h1_ref[...] + agg
    a = _leaky(jnp.dot(z, W1_ref[...], preferred_element_type=jnp.float32)
               + b1_ref[...])
    h2 = _leaky(jnp.dot(a, W2_ref[...], preferred_element_type=jnp.float32)
                + b2_ref[...])
    p_ref[...] = jnp.dot(h2, Wp_ref[...], preferred_element_type=jnp.float32)

  return pl.pallas_call(
      body,
      grid=(_GRID,),
      in_specs=[
          _row_spec(256), _pair_spec(128), _row_spec(128),
          _full_spec(256, 256), _full_spec(1, 256),
          _full_spec(256, 256), _full_spec(1, 256),
          _full_spec(256, 128),
          _full_spec(1, 1),
      ],
      out_specs=_row_spec(128),
      out_shape=jax.ShapeDtypeStruct((N, 128), jnp.float32),
  )(h1, s1, di, W1, b1, W2, b2, Wp, eps)


def _tc_layer2(p, s2, di, b1, W2, b2, Wo, bo, eps):
  def body(p_ref, s2_ref, di_ref, b1_ref, W2_ref, b2_ref, Wo_ref, bo_ref,
           eps_ref, no_ref, ne_ref):
    agg = (s2_ref[0] + s2_ref[1]) * di_ref[...]
    z = (1.0 + eps_ref[0, 0]) * p_ref[...] + agg + b1_ref[...]
    a = _leaky(z)
    ne = _leaky(jnp.dot(a, W2_ref[...], preferred_element_type=jnp.float32)
                + b2_ref[...])
    ne_ref[...] = ne
    no_ref[...] = (jnp.dot(ne, Wo_ref[...], preferred_element_type=jnp.float32)
                   + bo_ref[...])

  return pl.pallas_call(
      body,
      grid=(_GRID,),
      in_specs=[
          _row_spec(128), _pair_spec(128), _row_spec(128),
          _full_spec(1, 128),
          _full_spec(128, 128), _full_spec(1, 128),
          _full_spec(128, 128), _full_spec(1, 128),
          _full_spec(1, 1),
      ],
      out_specs=(_row_spec(128), _row_spec(128)),
      out_shape=(jax.ShapeDtypeStruct((N, 128), jnp.float32),
                 jax.ShapeDtypeStruct((N, 128), jnp.float32)),
  )(p, s2, di, b1, W2, b2, Wo, bo, eps)


def _chunk(a, workers, fill, ng=NG, ch=CH):
  per = E // workers
  a = a.reshape(workers, per)
  pad = (-per) % (ng * ch) + ng * ch
  a = jnp.concatenate(
      [a, jnp.full((workers, pad), fill, dtype=a.dtype)], axis=1)
  return a.reshape(workers, -1, ng, ch)


def kernel(x, edge_index,
           W1_0, b1_0, W2_0, b2_0,
           W1_1, b1_1, W2_1, b2_1,
           W1_2, b1_2, W2_2, b2_2,
           Wout, bout, eps0, eps1, eps2):
  src = edge_index[0].astype(jnp.int32)
  dst = edge_index[1].astype(jnp.int32)

  srcA_e2 = _chunk(src, NC * NS, 0, ng=3, ch=112)
  srcB_e2 = _chunk(src + N, NC * NS, N, ng=3, ch=112)
  dst_e2 = _chunk(dst, NC * NS, N, ng=3, ch=112)
  srcA_e = _chunk(src, NC * NS, 0)
  srcB_e = _chunk(src + N, NC * NS, N)
  dst_e = _chunk(dst, NC * NS, N)
  srcA = _chunk(src * 2, NS, 0, ng=3, ch=112)
  srcB = _chunk(src * 2 + 1, NS, 1, ng=3, ch=112)
  dst_c = _chunk(dst, NS, N, ng=3, ch=112)

  z128 = jnp.zeros((RB, 128), jnp.float32)

  def r2(b):
    return b.reshape(1, -1)

  def e2(e):
    return e.astype(jnp.float32).reshape(1, 1)

  s0, degs = _sc_agg(jnp.concatenate([x, x]), srcA_e, srcB_e, dst_e, z128,
                     with_deg=True, nbuf=2, la=1, ng=8)
  di = _tc_deg_prep(degs)
  h1 = _tc_layer0(x, s0, di, W1_0, r2(b1_0), W2_0, r2(b2_0), e2(eps0))

  s1 = _sc_agg(h1.reshape(2 * N, 128), srcA, srcB, dst_c, z128,
               with_deg=False, nbuf=3, la=2, ng=3, ch=112)[0]
  p = _tc_layer1(h1, s1, di, W1_1, r2(b1_1), W2_1, r2(b2_1), W1_2, e2(eps1))

  s2 = _sc_agg(jnp.concatenate([p, p]), srcA_e2, srcB_e2, dst_e2, z128,
               with_deg=False, nbuf=3, la=2, ng=3, ch=112)[0]
  Wo = jnp.pad(Wout, ((0, 0), (0, 128 - OUT_CH)))
  bo = jnp.pad(bout, (0, 128 - OUT_CH)).reshape(1, -1)
  n_out_pad, n_embed = _tc_layer2(p, s2, di, r2(b1_2), W2_2, r2(b2_2),
                                  Wo, bo, e2(eps2))
  return (n_out_pad[:, :OUT_CH], n_embed)


# speedup vs baseline: 1.5412x; 1.0071x over previous
import jax
import jax.numpy as jnp
from jax import lax
from jax.experimental import pallas as pl
from jax.experimental.pallas import tpu as pltpu
from jax.experimental.pallas import tpu_sc as plsc

N = 10000
E = 320000
NC = 2
NS = 16
CH = 128
NG = 8
NT = N + 8
RB = 624
RB_EX0 = NS * RB
HR = 80
OUT_CH = 2

_MESH = plsc.VectorSubcoreMesh(
    core_axis_name="c", subcore_axis_name="s", num_cores=NC, num_subcores=NS)


def _leaky(v):
  return jnp.where(v >= 0, v, 0.01 * v)


def _sc_agg(table, srcA, srcB, dst_g, z128, with_deg, nbuf=2, la=1, ng=NG,
            ch=CH):
  assert ng % nbuf == 0 and 0 < la < nbuf and la <= ng
  edge_split = srcA.shape[0] == NC * NS
  n_g = srcA.shape[1] - 1

  out_type = [jax.ShapeDtypeStruct((NC, N, 128), jnp.float32)]
  if with_deg:
    out_type.append(jax.ShapeDtypeStruct((NC * NS, HR, 128), jnp.float32))

  scratch = [
      pltpu.VMEM((2, ng, ch), jnp.int32),
      pltpu.VMEM((2, ng, ch), jnp.int32),
  ]
  scratch += [pltpu.VMEM((ch, 128), jnp.float32) for _ in range(nbuf)]
  if with_deg:
    scratch.append(pltpu.VMEM((HR, 128), jnp.float32))
  scratch.append(pltpu.VMEM_SHARED((NT, 128), jnp.float32))
  scratch += [pltpu.SemaphoreType.DMA] * (nbuf + 1)

  def body(table_h, srcA_h, srcB_h, dst_h, z128_h, *rest):
    if with_deg:
      out_h, deg_h = rest[0], rest[1]
      rest = rest[2:]
    else:
      out_h = rest[0]
      deg_h = None
      rest = rest[1:]
    src_v, dst_v = rest[0], rest[1]
    bufs = rest[2:2 + nbuf]
    rest = rest[2 + nbuf:]
    if with_deg:
      hist_v = rest[0]
      rest = rest[1:]
    else:
      hist_v = None
    agg_s = rest[0]
    gsems = rest[1:1 + nbuf]
    semi = rest[1 + nbuf]

    c = lax.axis_index("c")
    s = lax.axis_index("s")
    w = c * NS + s if edge_split else s
    r0 = s * RB

    def fetch_idx(g, slot):
      @pl.when(c == 0)
      def _():
        pltpu.async_copy(srcA_h.at[w, g], src_v.at[slot], semi)

      @pl.when(c == 1)
      def _():
        pltpu.async_copy(srcB_h.at[w, g], src_v.at[slot], semi)

      pltpu.async_copy(dst_h.at[w, g], dst_v.at[slot], semi)

    def wait_idx(g, slot):
      pltpu.make_async_copy(srcA_h.at[w, g], src_v.at[slot], semi).wait()
      pltpu.make_async_copy(dst_h.at[w, g], dst_v.at[slot], semi).wait()

    def start_gather(slot, k, b):
      pltpu.async_copy(table_h.at[src_v.at[slot, k]], bufs[b], gsems[b])

    def wait_gather(slot, k, b):
      pltpu.make_async_copy(table_h.at[src_v.at[slot, k]], bufs[b],
                            gsems[b]).wait()

    fetch_idx(0, 0)
    wait_idx(0, 0)
    for j in range(la):
      start_gather(0, j, j)

    zb = bufs[la]
    pltpu.sync_copy(z128_h.at[pl.ds(0, ch)], zb)
    for i in range(RB // ch):
      pltpu.sync_copy(zb, agg_s.at[pl.ds(r0 + i * ch, ch)])
    if RB % ch:
      pltpu.sync_copy(zb.at[pl.ds(0, RB % ch)],
                      agg_s.at[pl.ds(r0 + (RB // ch) * ch, RB % ch)])
    if with_deg:
      pltpu.sync_copy(z128_h.at[pl.ds(0, HR)], hist_v)

    @pl.when(s == NS - 1)
    def _():
      pltpu.sync_copy(zb.at[pl.ds(0, NT - RB_EX0)],
                      agg_s.at[pl.ds(RB_EX0, NT - RB_EX0)])

    plsc.subcore_barrier()

    ones16 = jnp.ones((16,), jnp.float32)

    @pl.loop(0, n_g)
    def _(g):
      p = lax.rem(g, 2)
      fetch_idx(g + 1, 1 - p)
      for k in range(ng):
        b = k % nbuf
        if k + la < ng:
          start_gather(p, k + la, (k + la) % nbuf)
        else:
          if k == ng - la:
            wait_idx(g + 1, 1 - p)
          start_gather(1 - p, k + la - ng, (k + la) % nbuf)
        wait_gather(p, k, b)
        pltpu.sync_copy(bufs[b], agg_s.at[dst_v.at[p, k]], add=True)
        if with_deg:
          for q in range(ch // 16):
            v = dst_v[p, k, pl.ds(q * 16, 16)]
            plsc.addupdate_scatter(
                hist_v, [lax.shift_right_logical(v, 7),
                         lax.bitwise_and(v, 127)], ones16)

    p_dummy = lax.rem(n_g, 2)
    for j in range(la):
      wait_gather(p_dummy, j, j)

    plsc.subcore_barrier()
    pltpu.sync_copy(agg_s.at[pl.ds(r0, RB)], out_h.at[c, pl.ds(r0, RB)])
    if with_deg:
      pltpu.sync_copy(hist_v, deg_h.at[c * NS + s])

    @pl.when(s == NS - 1)
    def _():
      pltpu.sync_copy(agg_s.at[pl.ds(RB_EX0, N - RB_EX0)],
                      out_h.at[c, pl.ds(RB_EX0, N - RB_EX0)])

  fn = pl.kernel(
      body, out_type=tuple(out_type), mesh=_MESH, scratch_types=scratch,
      compiler_params=pltpu.CompilerParams(needs_layout_passes=False))
  return fn(table, srcA, srcB, dst_g, z128)


_R = 1000
_GRID = N // _R


def _row_spec(d):
  return pl.BlockSpec((_R, d), lambda i: (i, 0))


def _pair_spec(d):
  return pl.BlockSpec((NC, _R, d), lambda i: (0, i, 0))


def _full_spec(r, c):
  return pl.BlockSpec((r, c), lambda i: (0, 0))


def _tc_deg_prep(degs):
  def body(d_ref, o_ref):
    hs = jnp.sum(d_ref[...], axis=0)
    deg = hs.reshape(HR * 128)[:N]
    di = 1.0 / jnp.maximum(deg, 1.0)
    o_ref[...] = jnp.broadcast_to(di[:, None], (N, 128))

  return pl.pallas_call(
      body,
      grid=(1,),
      in_specs=[pl.BlockSpec((NC * NS, HR, 128), lambda i: (0, 0, 0))],
      out_specs=pl.BlockSpec((N, 128), lambda i: (0, 0)),
      out_shape=jax.ShapeDtypeStruct((N, 128), jnp.float32),
  )(degs)


def _tc_layer0(x, s0, di, W1, b1, W2, b2, eps):
  def body(x_ref, s0_ref, di_ref, W1_ref, b1_ref, W2_ref, b2_ref, eps_ref,
           h1_ref):
    agg = (s0_ref[0] + s0_ref[1]) * di_ref[...]
    z = (1.0 + eps_ref[0, 0]) * x_ref[...] + agg
    a = _leaky(jnp.dot(z, W1_ref[...], preferred_element_type=jnp.float32)
               + b1_ref[...])
    h1_ref[...] = _leaky(
        jnp.dot(a, W2_ref[...], preferred_element_type=jnp.float32)
        + b2_ref[...])

  return pl.pallas_call(
      body,
      grid=(_GRID,),
      in_specs=[
          _row_spec(128), _pair_spec(128), _row_spec(128),
          _full_spec(128, 256), _full_spec(1, 256),
          _full_spec(256, 256), _full_spec(1, 256),
          _full_spec(1, 1),
      ],
      out_specs=_row_spec(256),
      out_shape=jax.ShapeDtypeStruct((N, 256), jnp.float32),
  )(x, s0, di, W1, b1, W2, b2, eps)


def _tc_layer1(h1, s1, di, W1, b1, W2, b2, Wp, eps):
  def body(h1_ref, s1_ref, di_ref, W1_ref, b1_ref, W2_ref, b2_ref,
           Wp_ref, eps_ref, p_ref):
    agg = jnp.concatenate([s1_ref[0], s1_ref[1]], axis=1) * di_ref[...][:, :1]
    z = (1.0 + eps_ref[0, 0]) * h1_ref[...] + agg
    a = _leaky(jnp.dot(z, W1_ref[...], preferred_element_type=jnp.float32)
               + b1_ref[...])
    h2 = _leaky(jnp.dot(a, W2_ref[...], preferred_element_type=jnp.float32)
                + b2_ref[...])
    p_ref[...] = jnp.dot(h2, Wp_ref[...], preferred_element_type=jnp.float32)

  return pl.pallas_call(
      body,
      grid=(_GRID,),
      in_specs=[
          _row_spec(256), _pair_spec(128), _row_spec(128),
          _full_spec(256, 256), _full_spec(1, 256),
          _full_spec(256, 256), _full_spec(1, 256),
          _full_spec(256, 128),
          _full_spec(1, 1),
      ],
      out_specs=_row_spec(128),
      out_shape=jax.ShapeDtypeStruct((N, 128), jnp.float32),
  )(h1, s1, di, W1, b1, W2, b2, Wp, eps)


def _tc_layer2(p, s2, di, b1, W2, b2, Wo, bo, eps):
  def body(p_ref, s2_ref, di_ref, b1_ref, W2_ref, b2_ref, Wo_ref, bo_ref,
           eps_ref, no_ref, ne_ref):
    agg = (s2_ref[0] + s2_ref[1]) * di_ref[...]
    z = (1.0 + eps_ref[0, 0]) * p_ref[...] + agg + b1_ref[...]
    a = _leaky(z)
    ne = _leaky(jnp.dot(a, W2_ref[...], preferred_element_type=jnp.float32)
                + b2_ref[...])
    ne_ref[...] = ne
    no_ref[...] = (jnp.dot(ne, Wo_ref[...], preferred_element_type=jnp.float32)
                   + bo_ref[...])

  return pl.pallas_call(
      body,
      grid=(_GRID,),
      in_specs=[
          _row_spec(128), _pair_spec(128), _row_spec(128),
          _full_spec(1, 128),
          _full_spec(128, 128), _full_spec(1, 128),
          _full_spec(128, 128), _full_spec(1, 128),
          _full_spec(1, 1),
      ],
      out_specs=(_row_spec(128), _row_spec(128)),
      out_shape=(jax.ShapeDtypeStruct((N, 128), jnp.float32),
                 jax.ShapeDtypeStruct((N, 128), jnp.float32)),
  )(p, s2, di, b1, W2, b2, Wo, bo, eps)


def _chunk(a, workers, fill, ng=NG, ch=CH):
  per = E // workers
  a = a.reshape(workers, per)
  pad = (-per) % (ng * ch) + ng * ch
  a = jnp.concatenate(
      [a, jnp.full((workers, pad), fill, dtype=a.dtype)], axis=1)
  return a.reshape(workers, -1, ng, ch)


def kernel(x, edge_index,
           W1_0, b1_0, W2_0, b2_0,
           W1_1, b1_1, W2_1, b2_1,
           W1_2, b1_2, W2_2, b2_2,
           Wout, bout, eps0, eps1, eps2):
  src = edge_index[0].astype(jnp.int32)
  dst = edge_index[1].astype(jnp.int32)

  srcA_e2 = _chunk(src, NC * NS, 0, ng=3, ch=112)
  srcB_e2 = _chunk(src + N, NC * NS, N, ng=3, ch=112)
  dst_e2 = _chunk(dst, NC * NS, N, ng=3, ch=112)
  srcA_e = _chunk(src, NC * NS, 0)
  srcB_e = _chunk(src + N, NC * NS, N)
  dst_e = _chunk(dst, NC * NS, N)
  srcA = _chunk(src * 2, NS, 0, ng=3, ch=112)
  srcB = _chunk(src * 2 + 1, NS, 1, ng=3, ch=112)
  dst_c = _chunk(dst, NS, N, ng=3, ch=112)

  z128 = jnp.zeros((RB, 128), jnp.float32)

  def r2(b):
    return b.reshape(1, -1)

  def e2(e):
    return e.astype(jnp.float32).reshape(1, 1)

  s0, degs = _sc_agg(jnp.concatenate([x, x]), srcA_e, srcB_e, dst_e, z128,
                     with_deg=True, nbuf=2, la=1, ng=8)
  di = _tc_deg_prep(degs)
  h1 = _tc_layer0(x, s0, di, W1_0, r2(b1_0), W2_0, r2(b2_0), e2(eps0))

  s1 = _sc_agg(h1.reshape(2 * N, 128), srcA, srcB, dst_c, z128,
               with_deg=False, nbuf=3, la=2, ng=3, ch=112)[0]
  p = _tc_layer1(h1, s1, di, W1_1, r2(b1_1), W2_1, r2(b2_1), W1_2, e2(eps1))

  s2 = _sc_agg(jnp.concatenate([p, p]), srcA_e2, srcB_e2, dst_e2, z128,
               with_deg=False, nbuf=3, la=2, ng=3, ch=112)[0]
  Wo = jnp.pad(Wout, ((0, 0), (0, 128 - OUT_CH)))
  bo = jnp.pad(bout, (0, 128 - OUT_CH)).reshape(1, -1)
  n_out_pad, n_embed = _tc_layer2(p, s2, di, r2(b1_2), W2_2, r2(b2_2),
                                  Wo, bo, e2(eps2))
  return (n_out_pad[:, :OUT_CH], n_embed)


# speedup vs baseline: 1.5545x; 1.0086x over previous
import jax
import jax.numpy as jnp
from jax import lax
from jax.experimental import pallas as pl
from jax.experimental.pallas import tpu as pltpu
from jax.experimental.pallas import tpu_sc as plsc

N = 10000
E = 320000
NC = 2
NS = 16
CH = 128
NG = 8
NT = N + 8
RB = 624
RB_EX0 = NS * RB
HR = 80
OUT_CH = 2

_MESH = plsc.VectorSubcoreMesh(
    core_axis_name="c", subcore_axis_name="s", num_cores=NC, num_subcores=NS)


def _leaky(v):
  return jnp.where(v >= 0, v, 0.01 * v)


def _sc_agg(table, srcA, srcB, dst_g, z128, with_deg, nbuf=2, la=1, ng=NG,
            ch=CH):
  assert ng % nbuf == 0 and 0 < la < nbuf and la <= ng
  edge_split = srcA.shape[0] == NC * NS
  n_g = srcA.shape[1] - 1

  out_type = [jax.ShapeDtypeStruct((NC, N, 128), jnp.float32)]
  if with_deg:
    out_type.append(jax.ShapeDtypeStruct((NC * NS, HR, 128), jnp.float32))

  scratch = [
      pltpu.VMEM((2, ng, ch), jnp.int32),
      pltpu.VMEM((2, ng, ch), jnp.int32),
  ]
  scratch += [pltpu.VMEM((ch, 128), jnp.float32) for _ in range(nbuf)]
  if with_deg:
    scratch.append(pltpu.VMEM((HR, 128), jnp.float32))
  scratch.append(pltpu.VMEM_SHARED((NT, 128), jnp.float32))
  scratch += [pltpu.SemaphoreType.DMA] * (nbuf + 1)

  def body(table_h, srcA_h, srcB_h, dst_h, z128_h, *rest):
    if with_deg:
      out_h, deg_h = rest[0], rest[1]
      rest = rest[2:]
    else:
      out_h = rest[0]
      deg_h = None
      rest = rest[1:]
    src_v, dst_v = rest[0], rest[1]
    bufs = rest[2:2 + nbuf]
    rest = rest[2 + nbuf:]
    if with_deg:
      hist_v = rest[0]
      rest = rest[1:]
    else:
      hist_v = None
    agg_s = rest[0]
    gsems = rest[1:1 + nbuf]
    semi = rest[1 + nbuf]

    c = lax.axis_index("c")
    s = lax.axis_index("s")
    w = c * NS + s if edge_split else s
    r0 = s * RB

    def fetch_idx(g, slot):
      @pl.when(c == 0)
      def _():
        pltpu.async_copy(srcA_h.at[w, g], src_v.at[slot], semi)

      @pl.when(c == 1)
      def _():
        pltpu.async_copy(srcB_h.at[w, g], src_v.at[slot], semi)

      pltpu.async_copy(dst_h.at[w, g], dst_v.at[slot], semi)

    def wait_idx(g, slot):
      pltpu.make_async_copy(srcA_h.at[w, g], src_v.at[slot], semi).wait()
      pltpu.make_async_copy(dst_h.at[w, g], dst_v.at[slot], semi).wait()

    def start_gather(slot, k, b):
      pltpu.async_copy(table_h.at[src_v.at[slot, k]], bufs[b], gsems[b])

    def wait_gather(slot, k, b):
      pltpu.make_async_copy(table_h.at[src_v.at[slot, k]], bufs[b],
                            gsems[b]).wait()

    fetch_idx(0, 0)
    wait_idx(0, 0)
    for j in range(la):
      start_gather(0, j, j)

    zb = bufs[la]
    pltpu.sync_copy(z128_h.at[pl.ds(0, ch)], zb)
    for i in range(RB // ch):
      pltpu.sync_copy(zb, agg_s.at[pl.ds(r0 + i * ch, ch)])
    if RB % ch:
      pltpu.sync_copy(zb.at[pl.ds(0, RB % ch)],
                      agg_s.at[pl.ds(r0 + (RB // ch) * ch, RB % ch)])
    if with_deg:
      pltpu.sync_copy(z128_h.at[pl.ds(0, HR)], hist_v)

    @pl.when(s == NS - 1)
    def _():
      pltpu.sync_copy(zb.at[pl.ds(0, NT - RB_EX0)],
                      agg_s.at[pl.ds(RB_EX0, NT - RB_EX0)])

    plsc.subcore_barrier()

    ones16 = jnp.ones((16,), jnp.float32)

    @pl.loop(0, n_g)
    def _(g):
      p = lax.rem(g, 2)
      fetch_idx(g + 1, 1 - p)
      for k in range(ng):
        b = k % nbuf
        if k + la < ng:
          start_gather(p, k + la, (k + la) % nbuf)
        else:
          if k == ng - la:
            wait_idx(g + 1, 1 - p)
          start_gather(1 - p, k + la - ng, (k + la) % nbuf)
        wait_gather(p, k, b)
        pltpu.sync_copy(bufs[b], agg_s.at[dst_v.at[p, k]], add=True)
        if with_deg:
          for q in range(ch // 16):
            v = dst_v[p, k, pl.ds(q * 16, 16)]
            plsc.addupdate_scatter(
                hist_v, [lax.shift_right_logical(v, 7),
                         lax.bitwise_and(v, 127)], ones16)

    p_dummy = lax.rem(n_g, 2)
    for j in range(la):
      wait_gather(p_dummy, j, j)

    plsc.subcore_barrier()
    pltpu.sync_copy(agg_s.at[pl.ds(r0, RB)], out_h.at[c, pl.ds(r0, RB)])
    if with_deg:
      pltpu.sync_copy(hist_v, deg_h.at[c * NS + s])

    @pl.when(s == NS - 1)
    def _():
      pltpu.sync_copy(agg_s.at[pl.ds(RB_EX0, N - RB_EX0)],
                      out_h.at[c, pl.ds(RB_EX0, N - RB_EX0)])

  fn = pl.kernel(
      body, out_type=tuple(out_type), mesh=_MESH, scratch_types=scratch,
      compiler_params=pltpu.CompilerParams(needs_layout_passes=False))
  return fn(table, srcA, srcB, dst_g, z128)


_R = 1000
_GRID = N // _R


def _row_spec(d):
  return pl.BlockSpec((_R, d), lambda i: (i, 0))


def _pair_spec(d):
  return pl.BlockSpec((NC, _R, d), lambda i: (0, i, 0))


def _full_spec(r, c):
  return pl.BlockSpec((r, c), lambda i: (0, 0))


def _tc_deg_prep(degs):
  def body(d_ref, o_ref):
    hs = jnp.sum(d_ref[...], axis=0)
    deg = hs.reshape(HR * 128)[:N]
    di = 1.0 / jnp.maximum(deg, 1.0)
    o_ref[...] = jnp.broadcast_to(di[:, None], (N, 128))

  return pl.pallas_call(
      body,
      grid=(1,),
      in_specs=[pl.BlockSpec((NC * NS, HR, 128), lambda i: (0, 0, 0))],
      out_specs=pl.BlockSpec((N, 128), lambda i: (0, 0)),
      out_shape=jax.ShapeDtypeStruct((N, 128), jnp.float32),
  )(degs)


def _tc_layer0(x, s0, di, W1, b1, W2, b2, eps):
  def body(x_ref, s0_ref, di_ref, W1_ref, b1_ref, W2_ref, b2_ref, eps_ref,
           h1_ref):
    agg = (s0_ref[0] + s0_ref[1]) * di_ref[...]
    z = (1.0 + eps_ref[0, 0]) * x_ref[...] + agg
    a = _leaky(jnp.dot(z, W1_ref[...], preferred_element_type=jnp.float32)
               + b1_ref[...])
    h1_ref[...] = _leaky(
        jnp.dot(a, W2_ref[...], preferred_element_type=jnp.float32)
        + b2_ref[...])

  return pl.pallas_call(
      body,
      grid=(_GRID,),
      in_specs=[
          _row_spec(128), _pair_spec(128), _row_spec(128),
          _full_spec(128, 256), _full_spec(1, 256),
          _full_spec(256, 256), _full_spec(1, 256),
          _full_spec(1, 1),
      ],
      out_specs=_row_spec(256),
      out_shape=jax.ShapeDtypeStruct((N, 256), jnp.float32),
  )(x, s0, di, W1, b1, W2, b2, eps)


def _tc_layer1(h1, s1, di, W1, b1, W2, b2, Wp, eps):
  def body(h1_ref, s1_ref, di_ref, W1_ref, b1_ref, W2_ref, b2_ref,
           Wp_ref, eps_ref, p_ref):
    agg = jnp.concatenate([s1_ref[0], s1_ref[1]], axis=1) * di_ref[...][:, :1]
    z = (1.0 + eps_ref[0, 0]) * h1_ref[...] + agg
    a = _leaky(jnp.dot(z, W1_ref[...], preferred_element_type=jnp.float32)
               + b1_ref[...])
    h2 = _leaky(jnp.dot(a, W2_ref[...], preferred_element_type=jnp.float32)
                + b2_ref[...])
    p = jnp.dot(h2, Wp_ref[...], preferred_element_type=jnp.float32)
    p_ref[0] = p
    p_ref[1] = p

  return pl.pallas_call(
      body,
      grid=(_GRID,),
      in_specs=[
          _row_spec(256), _pair_spec(128), _row_spec(128),
          _full_spec(256, 256), _full_spec(1, 256),
          _full_spec(256, 256), _full_spec(1, 256),
          _full_spec(256, 128),
          _full_spec(1, 1),
      ],
      out_specs=pl.BlockSpec((2, _R, 128), lambda i: (0, i, 0)),
      out_shape=jax.ShapeDtypeStruct((2, N, 128), jnp.float32),
  )(h1, s1, di, W1, b1, W2, b2, Wp, eps)


def _tc_layer2(p, s2, di, b1, W2, b2, Wo, bo, eps):
  def body(p_ref, s2_ref, di_ref, b1_ref, W2_ref, b2_ref, Wo_ref, bo_ref,
           eps_ref, no_ref, ne_ref):
    agg = (s2_ref[0] + s2_ref[1]) * di_ref[...]
    z = (1.0 + eps_ref[0, 0]) * p_ref[0] + agg + b1_ref[...]
    a = _leaky(z)
    ne = _leaky(jnp.dot(a, W2_ref[...], preferred_element_type=jnp.float32)
                + b2_ref[...])
    ne_ref[...] = ne
    no_ref[...] = (jnp.dot(ne, Wo_ref[...], preferred_element_type=jnp.float32)
                   + bo_ref[...])

  return pl.pallas_call(
      body,
      grid=(_GRID,),
      in_specs=[
          pl.BlockSpec((1, _R, 128), lambda i: (0, i, 0)),
          _pair_spec(128), _row_spec(128),
          _full_spec(1, 128),
          _full_spec(128, 128), _full_spec(1, 128),
          _full_spec(128, 128), _full_spec(1, 128),
          _full_spec(1, 1),
      ],
      out_specs=(_row_spec(128), _row_spec(128)),
      out_shape=(jax.ShapeDtypeStruct((N, 128), jnp.float32),
                 jax.ShapeDtypeStruct((N, 128), jnp.float32)),
  )(p, s2, di, b1, W2, b2, Wo, bo, eps)


def _chunk(a, workers, fill, ng=NG, ch=CH):
  per = E // workers
  a = a.reshape(workers, per)
  pad = (-per) % (ng * ch) + ng * ch
  a = jnp.concatenate(
      [a, jnp.full((workers, pad), fill, dtype=a.dtype)], axis=1)
  return a.reshape(workers, -1, ng, ch)


def kernel(x, edge_index,
           W1_0, b1_0, W2_0, b2_0,
           W1_1, b1_1, W2_1, b2_1,
           W1_2, b1_2, W2_2, b2_2,
           Wout, bout, eps0, eps1, eps2):
  src = edge_index[0].astype(jnp.int32)
  dst = edge_index[1].astype(jnp.int32)

  srcA_e2 = _chunk(src, NC * NS, 0, ng=3, ch=112)
  srcB_e2 = _chunk(src + N, NC * NS, N, ng=3, ch=112)
  dst_e2 = _chunk(dst, NC * NS, N, ng=3, ch=112)
  srcA_e = _chunk(src, NC * NS, 0)
  srcB_e = _chunk(src + N, NC * NS, N)
  dst_e = _chunk(dst, NC * NS, N)
  srcA = _chunk(src * 2, NS, 0, ng=3, ch=112)
  srcB = _chunk(src * 2 + 1, NS, 1, ng=3, ch=112)
  dst_c = _chunk(dst, NS, N, ng=3, ch=112)

  z128 = jnp.zeros((RB, 128), jnp.float32)

  def r2(b):
    return b.reshape(1, -1)

  def e2(e):
    return e.astype(jnp.float32).reshape(1, 1)

  s0, degs = _sc_agg(jnp.concatenate([x, x]), srcA_e, srcB_e, dst_e, z128,
                     with_deg=True, nbuf=2, la=1, ng=8)
  di = _tc_deg_prep(degs)
  h1 = _tc_layer0(x, s0, di, W1_0, r2(b1_0), W2_0, r2(b2_0), e2(eps0))

  s1 = _sc_agg(h1.reshape(2 * N, 128), srcA, srcB, dst_c, z128,
               with_deg=False, nbuf=3, la=2, ng=3, ch=112)[0]
  p2 = _tc_layer1(h1, s1, di, W1_1, r2(b1_1), W2_1, r2(b2_1), W1_2, e2(eps1))

  s2 = _sc_agg(p2.reshape(2 * N, 128), srcA_e2, srcB_e2, dst_e2, z128,
               with_deg=False, nbuf=3, la=2, ng=3, ch=112)[0]
  Wo = jnp.pad(Wout, ((0, 0), (0, 128 - OUT_CH)))
  bo = jnp.pad(bout, (0, 128 - OUT_CH)).reshape(1, -1)
  n_out_pad, n_embed = _tc_layer2(p2, s2, di, r2(b1_2), W2_2, r2(b2_2),
                                  Wo, bo, e2(eps2))
  return (n_out_pad[:, :OUT_CH], n_embed)
